# baseline (speedup 1.0000x reference)
import jax
import jax.numpy as jnp
from jax import lax
from jax.experimental import pallas as pl
from jax.experimental.pallas import tpu as pltpu
from jax.experimental.pallas import tpu_sc as plsc

N = 10000
E = 320000
D = 128
HID = 256
G = 64
SLOPE = 0.3
BN_EPS = 1e-5

NC = 2
NS = 16

NPAD = 10240
RPT = NPAD // NS
EPAD = 321536
EPT = EPAD // NS
CHUNK = 128
NCHUNKS = EPT // CHUNK
KC = 64
NKC = EPT // KC
CHUNK2 = 64
NCHUNKS2 = EPT // CHUNK2

_mesh = plsc.VectorSubcoreMesh(
    core_axis_name="c", subcore_axis_name="s", num_cores=NC, num_subcores=NS)

_DNUMS = lax.GatherDimensionNumbers(offset_dims=(), collapsed_slice_dims=(0,),
                                    start_index_map=(0,))


def _vgather(v, idx):
    return lax.gather(v, idx.reshape(16, 1), _DNUMS, (1,),
                      mode=lax.GatherScatterMode.PROMISE_IN_BOUNDS)


def _lanesum(v, io):
    s = v + _vgather(v, io ^ 8)
    s = s + _vgather(s, io ^ 4)
    s = s + _vgather(s, io ^ 2)
    return s + _vgather(s, io ^ 1)


def _splat(v, j, io):
    return _lanesum(jnp.where(io == j, v, 0.0), io)


def _sc_l1_logits(xlv, xrv, srcp, dstp, attr, z128):
    nh = 4

    def body(xlv_r, xrv_r, srcp_r, dstp_r, attr_r, z128_r, exh_r, denh_r,
             srcb, dstb, gsb, gdb, xlb, xrb, attb, exb, exrow, dens, sem):
        cid = lax.axis_index("c")
        sid = lax.axis_index("s")
        rsl = pl.ds(sid * RPT, RPT)
        pltpu.sync_copy(z128_r.at[rsl], dens.at[rsl])
        plsc.subcore_barrier()

        pltpu.sync_copy(attr_r.at[cid], attb)
        io = lax.iota(jnp.int32, 16)
        attv = [attb[pl.ds(16 * c, 16)] for c in range(8)]
        ebase0 = sid * EPT

        def chunk_body(k, carry):
            base = ebase0 + k * KC
            d1 = pltpu.async_copy(srcp_r.at[pl.ds(base, KC)], srcb, sem)
            d2 = pltpu.async_copy(dstp_r.at[pl.ds(base, KC)], dstb, sem)
            d1.wait()
            d2.wait()
            for i in range(KC // 16):
                sl = pl.ds(16 * i, 16)
                gsb[sl] = srcb[sl] * 2 + cid
                gdb[sl] = dstb[sl] * 2 + cid
            d1 = pltpu.async_copy(xlv_r.at[gsb], xlb, sem)
            d2 = pltpu.async_copy(xrv_r.at[gdb], xrb, sem)
            d1.wait()
            d2.wait()
            for g in range(KC // 16):
                def logit_body(j, accs):
                    j2 = g * 16 + j
                    new = []
                    for h in range(nh):
                        part = None
                        for q in range(2):
                            ch = 2 * h + q
                            sl = pl.ds(16 * ch, 16)
                            hv = xlb[j2, sl] + xrb[j2, sl]
                            ev = jnp.maximum(hv, SLOPE * hv)
                            t = ev * attv[ch]
                            part = t if part is None else part + t
                        tot = _lanesum(part, io)
                        new.append(jnp.where(io == j, tot, accs[h]))
                    return tuple(new)
                accs = lax.fori_loop(0, 16, logit_body,
                                     tuple(jnp.zeros((16,), jnp.float32)
                                           for _ in range(nh)))
                exs = [jnp.exp(a) for a in accs]
                for h in range(nh):
                    exb[pl.ds(h * KC + g * 16, 16)] = exs[h]

                for j in range(16):
                    j2 = g * 16 + j
                    for h in range(nh):
                        sp = _splat(exs[h], j, io)
                        exrow[j2, pl.ds(h * 32, 16)] = sp
                        exrow[j2, pl.ds(h * 32 + 16, 16)] = sp
            pltpu.sync_copy(exb, exh_r.at[cid, sid, k])
            pltpu.sync_copy(exrow, dens.at[dstb], add=True)
            return carry

        lax.fori_loop(0, NKC, chunk_body, 0)
        plsc.subcore_barrier()
        pltpu.sync_copy(dens.at[rsl], denh_r.at[cid, rsl])

    f = pl.kernel(
        body,
        out_type=[jax.ShapeDtypeStruct((NC, NS, NKC, 4 * KC), jnp.float32),
                  jax.ShapeDtypeStruct((NC, NPAD, 128), jnp.float32)],
        mesh=_mesh,
        scratch_types=[
            pltpu.VMEM((KC,), jnp.int32),
            pltpu.VMEM((KC,), jnp.int32),
            pltpu.VMEM((KC,), jnp.int32),
            pltpu.VMEM((KC,), jnp.int32),
            pltpu.VMEM((KC, 128), jnp.float32),
            pltpu.VMEM((KC, 128), jnp.float32),
            pltpu.VMEM((128,), jnp.float32),
            pltpu.VMEM((4 * KC,), jnp.float32),
            pltpu.VMEM((KC, 128), jnp.float32),
            pltpu.VMEM_SHARED((NPAD, 128), jnp.float32),
            pltpu.SemaphoreType.DMA,
        ],
    )
    return f(xlv, xrv, srcp, dstp, attr, z128)


def _sc_l1_messages(xlv, srcp, dstp, exh, denf, z128):
    nh = 4

    def body(xlv_r, srcp_r, dstp_r, exh_r, denf_r, z128_r, out_r,
             srcb, dstb0, dstb1, gsb, gdb, xlb, denb, exb, msgb0, msgb1,
             outs, sem, sem2):
        cid = lax.axis_index("c")
        sid = lax.axis_index("s")
        rsl = pl.ds(sid * RPT, RPT)
        pltpu.sync_copy(z128_r.at[rsl], outs.at[rsl])
        plsc.subcore_barrier()

        io = lax.iota(jnp.int32, 16)
        ebase0 = sid * EPT
        dstbs = [dstb0, dstb1]
        msgbs = [msgb0, msgb1]

        def pair_body(p, carry):
            for half in range(2):
                dstb = dstbs[half]
                msgb = msgbs[half]
                base = ebase0 + (2 * p + half) * KC

                @pl.when(p > 0)
                def _():
                    pltpu.make_async_copy(z128_r.at[pl.ds(0, KC)],
                                          msgb, sem2).wait()

                d1 = pltpu.async_copy(srcp_r.at[pl.ds(base, KC)], srcb, sem)
                d2 = pltpu.async_copy(dstp_r.at[pl.ds(base, KC)], dstb, sem)
                d3 = pltpu.async_copy(exh_r.at[cid, sid, 2 * p + half], exb, sem)
                d1.wait()
                d2.wait()
                for i in range(KC // 16):
                    sl = pl.ds(16 * i, 16)
                    gsb[sl] = srcb[sl] * 2 + cid
                    gdb[sl] = dstb[sl] + cid * NPAD
                d1 = pltpu.async_copy(xlv_r.at[gsb], xlb, sem)
                d2 = pltpu.async_copy(denf_r.at[gdb], denb, sem)
                d1.wait()
                d2.wait()
                d3.wait()
                for g in range(KC // 16):
                    exvs = [exb[pl.ds(h * KC + g * 16, 16)] for h in range(nh)]

                    for j in range(16):
                        j2 = g * 16 + j
                        for h in range(nh):
                            asp = (_splat(exvs[h], j, io)
                                   / (denb[j2, pl.ds(h * 32, 16)] + 1e-16))
                            for q in range(2):
                                sl = pl.ds((2 * h + q) * 16, 16)
                                msgb[j2, sl] = xlb[j2, sl] * asp
                pltpu.async_copy(msgb, outs.at[dstb], sem2, add=True)
            return carry

        lax.fori_loop(0, NKC // 2, pair_body, 0)
        for half in range(2):
            pltpu.make_async_copy(z128_r.at[pl.ds(0, KC)],
                                  msgbs[half], sem2).wait()
        plsc.subcore_barrier()
        pltpu.sync_copy(outs.at[rsl], out_r.at[cid, rsl])

    f = pl.kernel(
        body,
        out_type=[jax.ShapeDtypeStruct((NC, NPAD, 128), jnp.float32)],
        mesh=_mesh,
        scratch_types=[
            pltpu.VMEM((KC,), jnp.int32),
            pltpu.VMEM((KC,), jnp.int32),
            pltpu.VMEM((KC,), jnp.int32),
            pltpu.VMEM((KC,), jnp.int32),
            pltpu.VMEM((KC,), jnp.int32),
            pltpu.VMEM((KC, 128), jnp.float32),
            pltpu.VMEM((KC, 128), jnp.float32),
            pltpu.VMEM((4 * KC,), jnp.float32),
            pltpu.VMEM((KC, 128), jnp.float32),
            pltpu.VMEM((KC, 128), jnp.float32),
            pltpu.VMEM_SHARED((NPAD, 128), jnp.float32),
            pltpu.SemaphoreType.DMA,
            pltpu.SemaphoreType.DMA,
        ],
    )
    return f(xlv, srcp, dstp, exh, denf, z128)


def _sc_l2_plogits(xlv, xrv, srcp, dstp, attr):
    def body(xlv_r, xrv_r, srcp_r, dstp_r, attr_r, ph_r,
             srcb, dstb, gsb, gdb, xlb, xrb, attb, pb, sem):
        cid = lax.axis_index("c")
        sid = lax.axis_index("s")
        pltpu.sync_copy(attr_r.at[cid], attb)
        io = lax.iota(jnp.int32, 16)
        attv = [attb[pl.ds(16 * c, 16)] for c in range(8)]
        ebase0 = sid * EPT

        def chunk_body(k, carry):
            base = ebase0 + k * CHUNK
            d1 = pltpu.async_copy(srcp_r.at[pl.ds(base, CHUNK)], srcb, sem)
            d2 = pltpu.async_copy(dstp_r.at[pl.ds(base, CHUNK)], dstb, sem)
            d1.wait()
            d2.wait()
            for i in range(CHUNK // 16):
                sl = pl.ds(16 * i, 16)
                gsb[sl] = srcb[sl] * 2 + cid
                gdb[sl] = dstb[sl] * 2 + cid
            d1 = pltpu.async_copy(xlv_r.at[gsb], xlb, sem)
            d2 = pltpu.async_copy(xrv_r.at[gdb], xrb, sem)
            d1.wait()
            d2.wait()
            for g in range(CHUNK // 16):
                def logit_body(j, acc):
                    j2 = g * 16 + j
                    part = None
                    for ch in range(8):
                        sl = pl.ds(16 * ch, 16)
                        hv = xlb[j2, sl] + xrb[j2, sl]
                        ev = jnp.maximum(hv, SLOPE * hv)
                        t = ev * attv[ch]
                        part = t if part is None else part + t
                    tot = _lanesum(part, io)
                    return jnp.where(io == j, tot, acc)
                acc = lax.fori_loop(0, 16, logit_body, jnp.zeros((16,), jnp.float32))
                pb[pl.ds(g * 16, 16)] = acc
            pltpu.sync_copy(pb, ph_r.at[cid, pl.ds(base, CHUNK)])
            return carry

        lax.fori_loop(0, NCHUNKS, chunk_body, 0)

    f = pl.kernel(
        body,
        out_type=[jax.ShapeDtypeStruct((NC, EPAD), jnp.float32)],
        mesh=_mesh,
        scratch_types=[
            pltpu.VMEM((CHUNK,), jnp.int32),
            pltpu.VMEM((CHUNK,), jnp.int32),
            pltpu.VMEM((CHUNK,), jnp.int32),
            pltpu.VMEM((CHUNK,), jnp.int32),
            pltpu.VMEM((CHUNK, 128), jnp.float32),
            pltpu.VMEM((CHUNK, 128), jnp.float32),
            pltpu.VMEM((128,), jnp.float32),
            pltpu.VMEM((CHUNK,), jnp.float32),
            pltpu.SemaphoreType.DMA,
        ],
    )
    return f(xlv, xrv, srcp, dstp, attr)


def _sc_l2_combine(ph, dstp, z128):
    def body(ph_r, dstp_r, z128_r, ex2_r, den2_r,
             p0b, p1b, exb, dstb, exrow, dens, sem):
        cid = lax.axis_index("c")
        sid = lax.axis_index("s")
        rsl = pl.ds(sid * RPT, RPT)
        pltpu.sync_copy(z128_r.at[rsl], dens.at[rsl])
        plsc.subcore_barrier()
        ebase0 = sid * EPT

        def chunk_body(k, carry):
            base = ebase0 + k * CHUNK2
            d1 = pltpu.async_copy(ph_r.at[0, pl.ds(base, CHUNK2)], p0b, sem)
            d2 = pltpu.async_copy(ph_r.at[1, pl.ds(base, CHUNK2)], p1b, sem)
            d3 = pltpu.async_copy(dstp_r.at[pl.ds(base, CHUNK2)], dstb, sem)
            d1.wait()
            d2.wait()
            d3.wait()
            exvs = []
            for i in range(CHUNK2 // 16):
                sl = pl.ds(16 * i, 16)
                ev = jnp.exp(p0b[sl] + p1b[sl])
                exb[sl] = ev
                exvs.append(ev)
            io = lax.iota(jnp.int32, 16)
            for g in range(CHUNK2 // 16):
                exv = exvs[g]

                for j in range(16):
                    sp = _splat(exv, j, io)
                    exrow[g * 16 + j, pl.ds(0, 16)] = sp
            pltpu.sync_copy(exb, ex2_r.at[cid, pl.ds(base, CHUNK2)])
            pltpu.sync_copy(exrow, dens.at[dstb], add=True)
            return carry

        lax.fori_loop(0, NCHUNKS2, chunk_body, 0)
        plsc.subcore_barrier()
        pltpu.sync_copy(dens.at[rsl], den2_r.at[cid, rsl])

    f = pl.kernel(
        body,
        out_type=[jax.ShapeDtypeStruct((NC, EPAD), jnp.float32),
                  jax.ShapeDtypeStruct((NC, NPAD, 128), jnp.float32)],
        mesh=_mesh,
        scratch_types=[
            pltpu.VMEM((CHUNK2,), jnp.float32),
            pltpu.VMEM((CHUNK2,), jnp.float32),
            pltpu.VMEM((CHUNK2,), jnp.float32),
            pltpu.VMEM((CHUNK2,), jnp.int32),
            pltpu.VMEM((CHUNK2, 128), jnp.float32),
            pltpu.VMEM_SHARED((NPAD, 128), jnp.float32),
            pltpu.SemaphoreType.DMA,
        ],
    )
    return f(ph, dstp, z128)


def _sc_l2_messages(xlv, srcp, dstp, ex2, denf, z128):
    def body(xlv_r, srcp_r, dstp_r, ex2_r, denf_r, z128_r, out_r,
             srcb, dstb, gsb, gdb, xlb, denb, exb, msgb, outs, sem):
        cid = lax.axis_index("c")
        sid = lax.axis_index("s")
        rsl = pl.ds(sid * RPT, RPT)
        pltpu.sync_copy(z128_r.at[rsl], outs.at[rsl])
        plsc.subcore_barrier()
        io = lax.iota(jnp.int32, 16)
        ebase0 = sid * EPT

        def chunk_body(k, carry):
            base = ebase0 + k * KC
            d1 = pltpu.async_copy(srcp_r.at[pl.ds(base, KC)], srcb, sem)
            d2 = pltpu.async_copy(dstp_r.at[pl.ds(base, KC)], dstb, sem)
            d3 = pltpu.async_copy(ex2_r.at[cid, pl.ds(base, KC)], exb, sem)
            d1.wait()
            d2.wait()
            for i in range(KC // 16):
                sl = pl.ds(16 * i, 16)
                gsb[sl] = srcb[sl] * 2 + cid
                gdb[sl] = dstb[sl] + cid * NPAD
            d1 = pltpu.async_copy(xlv_r.at[gsb], xlb, sem)
            d2 = pltpu.async_copy(denf_r.at[gdb], denb, sem)
            d1.wait()
            d2.wait()
            d3.wait()
            for g in range(KC // 16):
                exv = exb[pl.ds(g * 16, 16)]

                for j in range(16):
                    j2 = g * 16 + j
                    asp = (_splat(exv, j, io)
                           / (denb[j2, pl.ds(0, 16)] + 1e-16))
                    for ch in range(8):
                        sl = pl.ds(16 * ch, 16)
                        msgb[j2, sl] = xlb[j2, sl] * asp
            pltpu.sync_copy(msgb, outs.at[dstb], add=True)
            return carry

        lax.fori_loop(0, NKC, chunk_body, 0)
        plsc.subcore_barrier()
        pltpu.sync_copy(outs.at[rsl], out_r.at[cid, rsl])

    f = pl.kernel(
        body,
        out_type=[jax.ShapeDtypeStruct((NC, NPAD, 128), jnp.float32)],
        mesh=_mesh,
        scratch_types=[
            pltpu.VMEM((KC,), jnp.int32),
            pltpu.VMEM((KC,), jnp.int32),
            pltpu.VMEM((KC,), jnp.int32),
            pltpu.VMEM((KC,), jnp.int32),
            pltpu.VMEM((KC, 128), jnp.float32),
            pltpu.VMEM((KC, 128), jnp.float32),
            pltpu.VMEM((KC,), jnp.float32),
            pltpu.VMEM((KC, 128), jnp.float32),
            pltpu.VMEM_SHARED((NPAD, 128), jnp.float32),
            pltpu.SemaphoreType.DMA,
        ],
    )
    return f(xlv, srcp, dstp, ex2, denf, z128)


_ROWS_BLK = 1000


def _tc_mm1(x, Wl, Wr):
    def body(x_ref, wl_ref, wr_ref, o1_ref, o2_ref):
        xv = x_ref[...]
        o1_ref[...] = jnp.dot(xv, wl_ref[...], preferred_element_type=jnp.float32)
        o2_ref[...] = jnp.dot(xv, wr_ref[...], preferred_element_type=jnp.float32)

    nb = N // _ROWS_BLK
    return pl.pallas_call(
        body,
        grid=(nb,),
        in_specs=[
            pl.BlockSpec((_ROWS_BLK, D), lambda i: (i, 0)),
            pl.BlockSpec((D, HID), lambda i: (0, 0)),
            pl.BlockSpec((D, HID), lambda i: (0, 0)),
        ],
        out_specs=[
            pl.BlockSpec((_ROWS_BLK, HID), lambda i: (i, 0)),
            pl.BlockSpec((_ROWS_BLK, HID), lambda i: (i, 0)),
        ],
        out_shape=[jax.ShapeDtypeStruct((N, HID), jnp.float32),
                   jax.ShapeDtypeStruct((N, HID), jnp.float32)],
    )(x, Wl, Wr)


def _tc_mm2(acc1, sg, off, Wl, Wr):
    def body(a_ref, sg_ref, off_ref, wl_ref, wr_ref, o1_ref, o2_ref):
        h = jnp.maximum(a_ref[...] * sg_ref[...] + off_ref[...], 0.0)
        o1_ref[...] = jnp.dot(h, wl_ref[...], preferred_element_type=jnp.float32)
        o2_ref[...] = jnp.dot(h, wr_ref[...], preferred_element_type=jnp.float32)

    nb = N // _ROWS_BLK
    return pl.pallas_call(
        body,
        grid=(nb,),
        in_specs=[
            pl.BlockSpec((_ROWS_BLK, HID), lambda i: (i, 0)),
            pl.BlockSpec((1, HID), lambda i: (0, 0)),
            pl.BlockSpec((1, HID), lambda i: (0, 0)),
            pl.BlockSpec((HID, HID), lambda i: (0, 0)),
            pl.BlockSpec((HID, HID), lambda i: (0, 0)),
        ],
        out_specs=[
            pl.BlockSpec((_ROWS_BLK, HID), lambda i: (i, 0)),
            pl.BlockSpec((_ROWS_BLK, HID), lambda i: (i, 0)),
        ],
        out_shape=[jax.ShapeDtypeStruct((N, HID), jnp.float32),
                   jax.ShapeDtypeStruct((N, HID), jnp.float32)],
    )(acc1, sg, off, Wl, Wr)


def _tc_final(acc2, sg, off, gW1, gb1, gW2t, batch2, fW1, fb1, fW2t, fb2):
    def body(a_ref, sg_ref, off_ref, gw1_ref, gb1_ref, gw2_ref, b_ref,
             fw1_ref, fb1_ref, fw2_ref, fb2_ref, o_ref):
        h2 = jnp.maximum(a_ref[...] * sg_ref[...] + off_ref[...], 0.0)
        t = jnp.tanh(jnp.dot(h2, gw1_ref[...],
                             preferred_element_type=jnp.float32) + gb1_ref[...])
        gate = lax.dot_general(gw2_ref[...], t, (((1,), (1,)), ((), ())),
                               preferred_element_type=jnp.float32)
        gmax = jnp.max(gate)
        gate2 = jnp.exp(gate - gmax)
        mask = lax.broadcasted_iota(jnp.int32, (G, N), 0) == b_ref[...]
        ge = jnp.where(mask, gate2, 0.0)
        gdt = lax.dot_general(jnp.ones((1, N), jnp.float32), ge,
                              (((1,), (1,)), ((), ())),
                              preferred_element_type=jnp.float32)
        poolt = lax.dot_general(h2, ge, (((0,), (1,)), ((), ())),
                                preferred_element_type=jnp.float32)
        poolt = poolt * (1.0 / (gdt + 1e-16))
        r = jnp.maximum(
            lax.dot_general(poolt, fw1_ref[...], (((0,), (0,)), ((), ())),
                            preferred_element_type=jnp.float32)
            + fb1_ref[...], 0.0)
        o_ref[...] = lax.dot_general(fw2_ref[...], r, (((1,), (1,)), ((), ())),
                                     preferred_element_type=jnp.float32) + fb2_ref[0, 0]

    out_t = pl.pallas_call(
        body,
        out_shape=jax.ShapeDtypeStruct((1, G), jnp.float32),
    )(acc2, sg, off, gW1, gb1, gW2t, batch2, fW1, fb1, fW2t, fb2)
    return out_t.reshape(G, 1)


def kernel(x, edge_index, batch, Wl1, Wr1, att1, b1, Wl2, Wr2, att2, b2,
           bn_g, bn_b, gW1, gb1, gW2, fW1, fb1, fW2, fb2):
    src = edge_index[0]
    dst = edge_index[1]
    npad_e = EPAD - E
    srcp = jnp.concatenate([src, jnp.zeros((npad_e,), jnp.int32)])
    dstp = jnp.concatenate([dst, jnp.full((npad_e,), N, jnp.int32)])
    z128 = jnp.zeros((NPAD, 128), jnp.float32)

    inv = 1.0 / jnp.sqrt(1.0 + BN_EPS)
    sg = (bn_g * inv).reshape(1, HID)
    off1 = (b1 * sg[0] + bn_b).reshape(1, HID)
    off2 = (b2 * sg[0] + bn_b).reshape(1, HID)

    xl1, xr1 = _tc_mm1(x, Wl1, Wr1)
    xlv = xl1.reshape(2 * N, 128)
    xrv = xr1.reshape(2 * N, 128)
    exh, denh = _sc_l1_logits(xlv, xrv, srcp, dstp,
                              att1.reshape(NC, 128), z128)
    denf = denh.reshape(NC * NPAD, 128)
    (o1h,) = _sc_l1_messages(xlv, srcp, dstp, exh, denf, z128)
    acc1 = jnp.concatenate([o1h[0, :N], o1h[1, :N]], axis=1)

    xl2, xr2 = _tc_mm2(acc1, sg, off1, Wl2, Wr2)
    xl2v = xl2.reshape(2 * N, 128)
    xr2v = xr2.reshape(2 * N, 128)
    (ph,) = _sc_l2_plogits(xl2v, xr2v, srcp, dstp, att2.reshape(NC, 128))
    ex2, den2 = _sc_l2_combine(ph, dstp, z128)
    den2f = den2.reshape(NC * NPAD, 128)
    (o2h,) = _sc_l2_messages(xl2v, srcp, dstp, ex2, den2f, z128)
    acc2 = jnp.concatenate([o2h[0, :N], o2h[1, :N]], axis=1)

    out = _tc_final(acc2, sg, off2, gW1, gb1.reshape(1, HID),
                    gW2.reshape(1, HID), batch.reshape(1, N),
                    fW1, fb1.reshape(1, 100), fW2.reshape(1, 100),
                    fb2.reshape(1, 1))
    return out


# speedup vs baseline: 1.0756x; 1.0756x over previous
import jax
import jax.numpy as jnp
from jax import lax
from jax.experimental import pallas as pl
from jax.experimental.pallas import tpu as pltpu
from jax.experimental.pallas import tpu_sc as plsc

N = 10000
E = 320000
D = 128
HID = 256
G = 64
SLOPE = 0.3
BN_EPS = 1e-5

NC = 2
NS = 16

NPAD = 10240
RPT = NPAD // NS
EPAD = 321536
EPT = EPAD // NS
CHUNK = 128
NCHUNKS = EPT // CHUNK
KC = 64
NKC = EPT // KC
CHUNK2 = 64
NCHUNKS2 = EPT // CHUNK2

_mesh = plsc.VectorSubcoreMesh(
    core_axis_name="c", subcore_axis_name="s", num_cores=NC, num_subcores=NS)

_DNUMS = lax.GatherDimensionNumbers(offset_dims=(), collapsed_slice_dims=(0,),
                                    start_index_map=(0,))


def _vgather(v, idx):
    return lax.gather(v, idx.reshape(16, 1), _DNUMS, (1,),
                      mode=lax.GatherScatterMode.PROMISE_IN_BOUNDS)


def _lanesum(v, io):
    s = v + _vgather(v, io ^ 8)
    s = s + _vgather(s, io ^ 4)
    s = s + _vgather(s, io ^ 2)
    return s + _vgather(s, io ^ 1)


def _splat(v, j, io):
    return _lanesum(jnp.where(io == j, v, 0.0), io)


def _sc_l1_logits(xlv, xrv, srcp, dstp, attr, z128):
    nh = 4

    def body(xlv_r, xrv_r, srcp_r, dstp_r, attr_r, z128_r, exh_r, denh_r,
             srcb, dstb, gsb, gdb, xlb, xrb, attb, exb, exrow, dens, sem):
        cid = lax.axis_index("c")
        sid = lax.axis_index("s")
        rsl = pl.ds(sid * RPT, RPT)
        pltpu.sync_copy(z128_r.at[rsl], dens.at[rsl])
        plsc.subcore_barrier()

        pltpu.sync_copy(attr_r.at[cid], attb)
        io = lax.iota(jnp.int32, 16)
        attv = [attb[pl.ds(16 * c, 16)] for c in range(8)]
        ebase0 = sid * EPT

        def chunk_body(k, carry):
            base = ebase0 + k * KC
            d1 = pltpu.async_copy(srcp_r.at[pl.ds(base, KC)], srcb, sem)
            d2 = pltpu.async_copy(dstp_r.at[pl.ds(base, KC)], dstb, sem)
            d1.wait()
            d2.wait()
            for i in range(KC // 16):
                sl = pl.ds(16 * i, 16)
                gsb[sl] = srcb[sl] * 2 + cid
                gdb[sl] = dstb[sl] * 2 + cid
            d1 = pltpu.async_copy(xlv_r.at[gsb], xlb, sem)
            d2 = pltpu.async_copy(xrv_r.at[gdb], xrb, sem)
            d1.wait()
            d2.wait()
            for g in range(KC // 16):
                def logit_body(j, accs):
                    j2 = g * 16 + j
                    new = []
                    for h in range(nh):
                        part = None
                        for q in range(2):
                            ch = 2 * h + q
                            sl = pl.ds(16 * ch, 16)
                            hv = xlb[j2, sl] + xrb[j2, sl]
                            ev = jnp.maximum(hv, SLOPE * hv)
                            t = ev * attv[ch]
                            part = t if part is None else part + t
                        tot = _lanesum(part, io)
                        new.append(jnp.where(io == j, tot, accs[h]))
                    return tuple(new)
                accs = lax.fori_loop(0, 16, logit_body,
                                     tuple(jnp.zeros((16,), jnp.float32)
                                           for _ in range(nh)))
                exs = [jnp.exp(a) for a in accs]
                for h in range(nh):
                    exb[pl.ds(h * KC + g * 16, 16)] = exs[h]

                for j in range(16):
                    j2 = g * 16 + j
                    for h in range(nh):
                        sp = _splat(exs[h], j, io)
                        exrow[j2, pl.ds(h * 32, 16)] = sp
                        exrow[j2, pl.ds(h * 32 + 16, 16)] = sp
            pltpu.sync_copy(exb, exh_r.at[cid, sid, k])
            pltpu.sync_copy(exrow, dens.at[dstb], add=True)
            return carry

        lax.fori_loop(0, NKC, chunk_body, 0)
        plsc.subcore_barrier()
        pltpu.sync_copy(dens.at[rsl], denh_r.at[cid, rsl])

    f = pl.kernel(
        body,
        out_type=[jax.ShapeDtypeStruct((NC, NS, NKC, 4 * KC), jnp.float32),
                  jax.ShapeDtypeStruct((NC, NPAD, 128), jnp.float32)],
        mesh=_mesh,
        scratch_types=[
            pltpu.VMEM((KC,), jnp.int32),
            pltpu.VMEM((KC,), jnp.int32),
            pltpu.VMEM((KC,), jnp.int32),
            pltpu.VMEM((KC,), jnp.int32),
            pltpu.VMEM((KC, 128), jnp.float32),
            pltpu.VMEM((KC, 128), jnp.float32),
            pltpu.VMEM((128,), jnp.float32),
            pltpu.VMEM((4 * KC,), jnp.float32),
            pltpu.VMEM((KC, 128), jnp.float32),
            pltpu.VMEM_SHARED((NPAD, 128), jnp.float32),
            pltpu.SemaphoreType.DMA,
        ],
    )
    return f(xlv, xrv, srcp, dstp, attr, z128)


def _sc_l1_messages(xlv, srcp, dstp, exh, denf, z128):
    nh = 4

    def body(xlv_r, srcp_r, dstp_r, exh_r, denf_r, z128_r, out_r,
             srcb, dstb, gsb, gdb, xlb, denb, exb, msgb, outs, sem):
        cid = lax.axis_index("c")
        sid = lax.axis_index("s")
        rsl = pl.ds(sid * RPT, RPT)
        pltpu.sync_copy(z128_r.at[rsl], outs.at[rsl])
        plsc.subcore_barrier()

        io = lax.iota(jnp.int32, 16)
        ebase0 = sid * EPT

        def chunk_body(k, carry):
            base = ebase0 + k * KC
            d1 = pltpu.async_copy(srcp_r.at[pl.ds(base, KC)], srcb, sem)
            d2 = pltpu.async_copy(dstp_r.at[pl.ds(base, KC)], dstb, sem)
            d3 = pltpu.async_copy(exh_r.at[cid, sid, k], exb, sem)
            d1.wait()
            d2.wait()
            for i in range(KC // 16):
                sl = pl.ds(16 * i, 16)
                gsb[sl] = srcb[sl] * 2 + cid
                gdb[sl] = dstb[sl] + cid * NPAD
            d1 = pltpu.async_copy(xlv_r.at[gsb], xlb, sem)
            d2 = pltpu.async_copy(denf_r.at[gdb], denb, sem)
            d1.wait()
            d2.wait()
            d3.wait()
            for g in range(KC // 16):
                exvs = [exb[pl.ds(h * KC + g * 16, 16)] for h in range(nh)]

                for j in range(16):
                    j2 = g * 16 + j
                    for h in range(nh):
                        asp = (_splat(exvs[h], j, io)
                               / (denb[j2, pl.ds(h * 32, 16)] + 1e-16))
                        for q in range(2):
                            sl = pl.ds((2 * h + q) * 16, 16)
                            msgb[j2, sl] = xlb[j2, sl] * asp
            pltpu.sync_copy(msgb, outs.at[dstb], add=True)
            return carry

        lax.fori_loop(0, NKC, chunk_body, 0)
        plsc.subcore_barrier()
        pltpu.sync_copy(outs.at[rsl], out_r.at[cid, rsl])

    f = pl.kernel(
        body,
        out_type=[jax.ShapeDtypeStruct((NC, NPAD, 128), jnp.float32)],
        mesh=_mesh,
        scratch_types=[
            pltpu.VMEM((KC,), jnp.int32),
            pltpu.VMEM((KC,), jnp.int32),
            pltpu.VMEM((KC,), jnp.int32),
            pltpu.VMEM((KC,), jnp.int32),
            pltpu.VMEM((KC, 128), jnp.float32),
            pltpu.VMEM((KC, 128), jnp.float32),
            pltpu.VMEM((4 * KC,), jnp.float32),
            pltpu.VMEM((KC, 128), jnp.float32),
            pltpu.VMEM_SHARED((NPAD, 128), jnp.float32),
            pltpu.SemaphoreType.DMA,
        ],
    )
    return f(xlv, srcp, dstp, exh, denf, z128)


def _sc_l2_plogits(xlv, xrv, srcp, dstp, attr):
    def body(xlv_r, xrv_r, srcp_r, dstp_r, attr_r, ph_r,
             srcb, dstb, gsb, gdb, xlb, xrb, attb, pb, sem):
        cid = lax.axis_index("c")
        sid = lax.axis_index("s")
        pltpu.sync_copy(attr_r.at[cid], attb)
        io = lax.iota(jnp.int32, 16)
        attv = [attb[pl.ds(16 * c, 16)] for c in range(8)]
        ebase0 = sid * EPT

        def chunk_body(k, carry):
            base = ebase0 + k * CHUNK
            d1 = pltpu.async_copy(srcp_r.at[pl.ds(base, CHUNK)], srcb, sem)
            d2 = pltpu.async_copy(dstp_r.at[pl.ds(base, CHUNK)], dstb, sem)
            d1.wait()
            d2.wait()
            for i in range(CHUNK // 16):
                sl = pl.ds(16 * i, 16)
                gsb[sl] = srcb[sl] * 2 + cid
                gdb[sl] = dstb[sl] * 2 + cid
            d1 = pltpu.async_copy(xlv_r.at[gsb], xlb, sem)
            d2 = pltpu.async_copy(xrv_r.at[gdb], xrb, sem)
            d1.wait()
            d2.wait()
            for g in range(CHUNK // 16):
                def logit_body(j, acc):
                    j2 = g * 16 + j
                    part = None
                    for ch in range(8):
                        sl = pl.ds(16 * ch, 16)
                        hv = xlb[j2, sl] + xrb[j2, sl]
                        ev = jnp.maximum(hv, SLOPE * hv)
                        t = ev * attv[ch]
                        part = t if part is None else part + t
                    tot = _lanesum(part, io)
                    return jnp.where(io == j, tot, acc)
                acc = lax.fori_loop(0, 16, logit_body, jnp.zeros((16,), jnp.float32))
                pb[pl.ds(g * 16, 16)] = acc
            pltpu.sync_copy(pb, ph_r.at[cid, pl.ds(base, CHUNK)])
            return carry

        lax.fori_loop(0, NCHUNKS, chunk_body, 0)

    f = pl.kernel(
        body,
        out_type=[jax.ShapeDtypeStruct((NC, EPAD), jnp.float32)],
        mesh=_mesh,
        scratch_types=[
            pltpu.VMEM((CHUNK,), jnp.int32),
            pltpu.VMEM((CHUNK,), jnp.int32),
            pltpu.VMEM((CHUNK,), jnp.int32),
            pltpu.VMEM((CHUNK,), jnp.int32),
            pltpu.VMEM((CHUNK, 128), jnp.float32),
            pltpu.VMEM((CHUNK, 128), jnp.float32),
            pltpu.VMEM((128,), jnp.float32),
            pltpu.VMEM((CHUNK,), jnp.float32),
            pltpu.SemaphoreType.DMA,
        ],
    )
    return f(xlv, xrv, srcp, dstp, attr)


def _sc_l2_combine(ph, dstp, z128):
    def body(ph_r, dstp_r, z128_r, ex2_r, den2_r,
             p0b, p1b, exb, dstb, exrow, dens, sem):
        cid = lax.axis_index("c")
        sid = lax.axis_index("s")
        rsl = pl.ds(sid * RPT, RPT)
        pltpu.sync_copy(z128_r.at[rsl], dens.at[rsl])
        plsc.subcore_barrier()
        ebase0 = sid * EPT

        def chunk_body(k, carry):
            base = ebase0 + k * CHUNK2
            d1 = pltpu.async_copy(ph_r.at[0, pl.ds(base, CHUNK2)], p0b, sem)
            d2 = pltpu.async_copy(ph_r.at[1, pl.ds(base, CHUNK2)], p1b, sem)
            d3 = pltpu.async_copy(dstp_r.at[pl.ds(base, CHUNK2)], dstb, sem)
            d1.wait()
            d2.wait()
            d3.wait()
            exvs = []
            for i in range(CHUNK2 // 16):
                sl = pl.ds(16 * i, 16)
                ev = jnp.exp(p0b[sl] + p1b[sl])
                exb[sl] = ev
                exvs.append(ev)
            io = lax.iota(jnp.int32, 16)
            for g in range(CHUNK2 // 16):
                exv = exvs[g]

                for j in range(16):
                    sp = _splat(exv, j, io)
                    exrow[g * 16 + j, pl.ds(0, 16)] = sp
            pltpu.sync_copy(exb, ex2_r.at[cid, pl.ds(base, CHUNK2)])
            pltpu.sync_copy(exrow, dens.at[dstb], add=True)
            return carry

        lax.fori_loop(0, NCHUNKS2, chunk_body, 0)
        plsc.subcore_barrier()
        pltpu.sync_copy(dens.at[rsl], den2_r.at[cid, rsl])

    f = pl.kernel(
        body,
        out_type=[jax.ShapeDtypeStruct((NC, EPAD), jnp.float32),
                  jax.ShapeDtypeStruct((NC, NPAD, 128), jnp.float32)],
        mesh=_mesh,
        scratch_types=[
            pltpu.VMEM((CHUNK2,), jnp.float32),
            pltpu.VMEM((CHUNK2,), jnp.float32),
            pltpu.VMEM((CHUNK2,), jnp.float32),
            pltpu.VMEM((CHUNK2,), jnp.int32),
            pltpu.VMEM((CHUNK2, 128), jnp.float32),
            pltpu.VMEM_SHARED((NPAD, 128), jnp.float32),
            pltpu.SemaphoreType.DMA,
        ],
    )
    return f(ph, dstp, z128)


def _sc_l2_messages(xlv, srcp, dstp, ex2, denf, z128):
    def body(xlv_r, srcp_r, dstp_r, ex2_r, denf_r, z128_r, out_r,
             srcb, dstb, gsb, gdb, xlb, denb, exb, msgb, outs, sem):
        cid = lax.axis_index("c")
        sid = lax.axis_index("s")
        rsl = pl.ds(sid * RPT, RPT)
        pltpu.sync_copy(z128_r.at[rsl], outs.at[rsl])
        plsc.subcore_barrier()
        io = lax.iota(jnp.int32, 16)
        ebase0 = sid * EPT

        def chunk_body(k, carry):
            base = ebase0 + k * KC
            d1 = pltpu.async_copy(srcp_r.at[pl.ds(base, KC)], srcb, sem)
            d2 = pltpu.async_copy(dstp_r.at[pl.ds(base, KC)], dstb, sem)
            d3 = pltpu.async_copy(ex2_r.at[cid, pl.ds(base, KC)], exb, sem)
            d1.wait()
            d2.wait()
            for i in range(KC // 16):
                sl = pl.ds(16 * i, 16)
                gsb[sl] = srcb[sl] * 2 + cid
                gdb[sl] = dstb[sl] + cid * NPAD
            d1 = pltpu.async_copy(xlv_r.at[gsb], xlb, sem)
            d2 = pltpu.async_copy(denf_r.at[gdb], denb, sem)
            d1.wait()
            d2.wait()
            d3.wait()
            for g in range(KC // 16):
                exv = exb[pl.ds(g * 16, 16)]

                for j in range(16):
                    j2 = g * 16 + j
                    asp = (_splat(exv, j, io)
                           / (denb[j2, pl.ds(0, 16)] + 1e-16))
                    for ch in range(8):
                        sl = pl.ds(16 * ch, 16)
                        msgb[j2, sl] = xlb[j2, sl] * asp
            pltpu.sync_copy(msgb, outs.at[dstb], add=True)
            return carry

        lax.fori_loop(0, NKC, chunk_body, 0)
        plsc.subcore_barrier()
        pltpu.sync_copy(outs.at[rsl], out_r.at[cid, rsl])

    f = pl.kernel(
        body,
        out_type=[jax.ShapeDtypeStruct((NC, NPAD, 128), jnp.float32)],
        mesh=_mesh,
        scratch_types=[
            pltpu.VMEM((KC,), jnp.int32),
            pltpu.VMEM((KC,), jnp.int32),
            pltpu.VMEM((KC,), jnp.int32),
            pltpu.VMEM((KC,), jnp.int32),
            pltpu.VMEM((KC, 128), jnp.float32),
            pltpu.VMEM((KC, 128), jnp.float32),
            pltpu.VMEM((KC,), jnp.float32),
            pltpu.VMEM((KC, 128), jnp.float32),
            pltpu.VMEM_SHARED((NPAD, 128), jnp.float32),
            pltpu.SemaphoreType.DMA,
        ],
    )
    return f(xlv, srcp, dstp, ex2, denf, z128)


_ROWS_BLK = 1000


def _tc_mm1(x, Wl, Wr):
    def body(x_ref, wl_ref, wr_ref, o1_ref, o2_ref):
        xv = x_ref[...]
        o1_ref[...] = jnp.dot(xv, wl_ref[...], preferred_element_type=jnp.float32)
        o2_ref[...] = jnp.dot(xv, wr_ref[...], preferred_element_type=jnp.float32)

    nb = N // _ROWS_BLK
    return pl.pallas_call(
        body,
        grid=(nb,),
        in_specs=[
            pl.BlockSpec((_ROWS_BLK, D), lambda i: (i, 0)),
            pl.BlockSpec((D, HID), lambda i: (0, 0)),
            pl.BlockSpec((D, HID), lambda i: (0, 0)),
        ],
        out_specs=[
            pl.BlockSpec((_ROWS_BLK, HID), lambda i: (i, 0)),
            pl.BlockSpec((_ROWS_BLK, HID), lambda i: (i, 0)),
        ],
        out_shape=[jax.ShapeDtypeStruct((N, HID), jnp.float32),
                   jax.ShapeDtypeStruct((N, HID), jnp.float32)],
    )(x, Wl, Wr)


def _tc_mm2(acc1, sg, off, Wl, Wr):
    def body(a_ref, sg_ref, off_ref, wl_ref, wr_ref, o1_ref, o2_ref):
        h = jnp.maximum(a_ref[...] * sg_ref[...] + off_ref[...], 0.0)
        o1_ref[...] = jnp.dot(h, wl_ref[...], preferred_element_type=jnp.float32)
        o2_ref[...] = jnp.dot(h, wr_ref[...], preferred_element_type=jnp.float32)

    nb = N // _ROWS_BLK
    return pl.pallas_call(
        body,
        grid=(nb,),
        in_specs=[
            pl.BlockSpec((_ROWS_BLK, HID), lambda i: (i, 0)),
            pl.BlockSpec((1, HID), lambda i: (0, 0)),
            pl.BlockSpec((1, HID), lambda i: (0, 0)),
            pl.BlockSpec((HID, HID), lambda i: (0, 0)),
            pl.BlockSpec((HID, HID), lambda i: (0, 0)),
        ],
        out_specs=[
            pl.BlockSpec((_ROWS_BLK, HID), lambda i: (i, 0)),
            pl.BlockSpec((_ROWS_BLK, HID), lambda i: (i, 0)),
        ],
        out_shape=[jax.ShapeDtypeStruct((N, HID), jnp.float32),
                   jax.ShapeDtypeStruct((N, HID), jnp.float32)],
    )(acc1, sg, off, Wl, Wr)


def _tc_final(acc2, sg, off, gW1, gb1, gW2t, batch2, fW1, fb1, fW2t, fb2):
    def body(a_ref, sg_ref, off_ref, gw1_ref, gb1_ref, gw2_ref, b_ref,
             fw1_ref, fb1_ref, fw2_ref, fb2_ref, o_ref):
        h2 = jnp.maximum(a_ref[...] * sg_ref[...] + off_ref[...], 0.0)
        t = jnp.tanh(jnp.dot(h2, gw1_ref[...],
                             preferred_element_type=jnp.float32) + gb1_ref[...])
        gate = lax.dot_general(gw2_ref[...], t, (((1,), (1,)), ((), ())),
                               preferred_element_type=jnp.float32)
        gmax = jnp.max(gate)
        gate2 = jnp.exp(gate - gmax)
        mask = lax.broadcasted_iota(jnp.int32, (G, N), 0) == b_ref[...]
        ge = jnp.where(mask, gate2, 0.0)
        gdt = lax.dot_general(jnp.ones((1, N), jnp.float32), ge,
                              (((1,), (1,)), ((), ())),
                              preferred_element_type=jnp.float32)
        poolt = lax.dot_general(h2, ge, (((0,), (1,)), ((), ())),
                                preferred_element_type=jnp.float32)
        poolt = poolt * (1.0 / (gdt + 1e-16))
        r = jnp.maximum(
            lax.dot_general(poolt, fw1_ref[...], (((0,), (0,)), ((), ())),
                            preferred_element_type=jnp.float32)
            + fb1_ref[...], 0.0)
        o_ref[...] = lax.dot_general(fw2_ref[...], r, (((1,), (1,)), ((), ())),
                                     preferred_element_type=jnp.float32) + fb2_ref[0, 0]

    out_t = pl.pallas_call(
        body,
        out_shape=jax.ShapeDtypeStruct((1, G), jnp.float32),
    )(acc2, sg, off, gW1, gb1, gW2t, batch2, fW1, fb1, fW2t, fb2)
    return out_t.reshape(G, 1)


def kernel(x, edge_index, batch, Wl1, Wr1, att1, b1, Wl2, Wr2, att2, b2,
           bn_g, bn_b, gW1, gb1, gW2, fW1, fb1, fW2, fb2):
    src = edge_index[0]
    dst = edge_index[1]
    npad_e = EPAD - E
    srcp = jnp.concatenate([src, jnp.zeros((npad_e,), jnp.int32)])
    dstp = jnp.concatenate([dst, jnp.full((npad_e,), N, jnp.int32)])
    z128 = jnp.zeros((NPAD, 128), jnp.float32)

    inv = 1.0 / jnp.sqrt(1.0 + BN_EPS)
    sg = (bn_g * inv).reshape(1, HID)
    off1 = (b1 * sg[0] + bn_b).reshape(1, HID)
    off2 = (b2 * sg[0] + bn_b).reshape(1, HID)

    xl1, xr1 = _tc_mm1(x, Wl1, Wr1)
    xlv = xl1.reshape(2 * N, 128)
    xrv = xr1.reshape(2 * N, 128)
    exh, denh = _sc_l1_logits(xlv, xrv, srcp, dstp,
                              att1.reshape(NC, 128), z128)
    denf = denh.reshape(NC * NPAD, 128)
    (o1h,) = _sc_l1_messages(xlv, srcp, dstp, exh, denf, z128)
    acc1 = jnp.concatenate([o1h[0, :N], o1h[1, :N]], axis=1)

    xl2, xr2 = _tc_mm2(acc1, sg, off1, Wl2, Wr2)
    xl2v = xl2.reshape(2 * N, 128)
    xr2v = xr2.reshape(2 * N, 128)
    (ph,) = _sc_l2_plogits(xl2v, xr2v, srcp, dstp, att2.reshape(NC, 128))
    ex2, den2 = _sc_l2_combine(ph, dstp, z128)
    den2f = den2.reshape(NC * NPAD, 128)
    (o2h,) = _sc_l2_messages(xl2v, srcp, dstp, ex2, den2f, z128)
    acc2 = jnp.concatenate([o2h[0, :N], o2h[1, :N]], axis=1)

    out = _tc_final(acc2, sg, off2, gW1, gb1.reshape(1, HID),
                    gW2.reshape(1, HID), batch.reshape(1, N),
                    fW1, fb1.reshape(1, 100), fW2.reshape(1, 100),
                    fb2.reshape(1, 1))
    return out


# speedup vs baseline: 1.0933x; 1.0165x over previous
import jax
import jax.numpy as jnp
from jax import lax
from jax.experimental import pallas as pl
from jax.experimental.pallas import tpu as pltpu
from jax.experimental.pallas import tpu_sc as plsc

N = 10000
E = 320000
D = 128
HID = 256
G = 64
SLOPE = 0.3
BN_EPS = 1e-5

NC = 2
NS = 16

NPAD = 10240
RPT = NPAD // NS
EPAD = 321536
EPT = EPAD // NS
CHUNK = 128
NCHUNKS = EPT // CHUNK
KC = 64
NKC = EPT // KC
CHUNK2 = 128
NCHUNKS2 = EPT // CHUNK2

_mesh = plsc.VectorSubcoreMesh(
    core_axis_name="c", subcore_axis_name="s", num_cores=NC, num_subcores=NS)

_DNUMS = lax.GatherDimensionNumbers(offset_dims=(), collapsed_slice_dims=(0,),
                                    start_index_map=(0,))


def _vgather(v, idx):
    return lax.gather(v, idx.reshape(16, 1), _DNUMS, (1,),
                      mode=lax.GatherScatterMode.PROMISE_IN_BOUNDS)


def _lanesum(v, io):
    s = v + _vgather(v, io ^ 8)
    s = s + _vgather(s, io ^ 4)
    s = s + _vgather(s, io ^ 2)
    return s + _vgather(s, io ^ 1)


def _splat(v, j, io):
    return _lanesum(jnp.where(io == j, v, 0.0), io)


def _sc_l1_logits(xlv, xrv, srcp, dstp, attr, z128):
    nh = 4

    def body(xlv_r, xrv_r, srcp_r, dstp_r, attr_r, z128_r, exh_r, denh_r,
             srcb, dstb, gsb, gdb, xlb, xrb, attb, exb, exrow, dens, sem):
        cid = lax.axis_index("c")
        sid = lax.axis_index("s")
        rsl = pl.ds(sid * RPT, RPT)
        pltpu.sync_copy(z128_r.at[rsl], dens.at[rsl])
        plsc.subcore_barrier()

        pltpu.sync_copy(attr_r.at[cid], attb)
        io = lax.iota(jnp.int32, 16)
        attv = [attb[pl.ds(16 * c, 16)] for c in range(8)]
        ebase0 = sid * EPT

        def chunk_body(k, carry):
            base = ebase0 + k * KC
            d1 = pltpu.async_copy(srcp_r.at[pl.ds(base, KC)], srcb, sem)
            d2 = pltpu.async_copy(dstp_r.at[pl.ds(base, KC)], dstb, sem)
            d1.wait()
            d2.wait()
            for i in range(KC // 16):
                sl = pl.ds(16 * i, 16)
                gsb[sl] = srcb[sl] * 2 + cid
                gdb[sl] = dstb[sl] * 2 + cid
            d1 = pltpu.async_copy(xlv_r.at[gsb], xlb, sem)
            d2 = pltpu.async_copy(xrv_r.at[gdb], xrb, sem)
            d1.wait()
            d2.wait()
            for g in range(KC // 16):
                def logit_body(j, accs):
                    j2 = g * 16 + j
                    new = []
                    for h in range(nh):
                        part = None
                        for q in range(2):
                            ch = 2 * h + q
                            sl = pl.ds(16 * ch, 16)
                            hv = xlb[j2, sl] + xrb[j2, sl]
                            ev = jnp.maximum(hv, SLOPE * hv)
                            t = ev * attv[ch]
                            part = t if part is None else part + t
                        tot = _lanesum(part, io)
                        new.append(jnp.where(io == j, tot, accs[h]))
                    return tuple(new)
                accs = lax.fori_loop(0, 16, logit_body,
                                     tuple(jnp.zeros((16,), jnp.float32)
                                           for _ in range(nh)))
                exs = [jnp.exp(a) for a in accs]
                for h in range(nh):
                    exb[pl.ds(h * KC + g * 16, 16)] = exs[h]

                for j in range(16):
                    j2 = g * 16 + j
                    for h in range(nh):
                        sp = _splat(exs[h], j, io)
                        exrow[j2, pl.ds(h * 32, 16)] = sp
                        exrow[j2, pl.ds(h * 32 + 16, 16)] = sp
            pltpu.sync_copy(exb, exh_r.at[cid, sid, k])
            pltpu.sync_copy(exrow, dens.at[dstb], add=True)
            return carry

        lax.fori_loop(0, NKC, chunk_body, 0)
        plsc.subcore_barrier()
        pltpu.sync_copy(dens.at[rsl], denh_r.at[cid, rsl])

    f = pl.kernel(
        body,
        out_type=[jax.ShapeDtypeStruct((NC, NS, NKC, 4 * KC), jnp.float32),
                  jax.ShapeDtypeStruct((NC, NPAD, 128), jnp.float32)],
        mesh=_mesh,
        scratch_types=[
            pltpu.VMEM((KC,), jnp.int32),
            pltpu.VMEM((KC,), jnp.int32),
            pltpu.VMEM((KC,), jnp.int32),
            pltpu.VMEM((KC,), jnp.int32),
            pltpu.VMEM((KC, 128), jnp.float32),
            pltpu.VMEM((KC, 128), jnp.float32),
            pltpu.VMEM((128,), jnp.float32),
            pltpu.VMEM((4 * KC,), jnp.float32),
            pltpu.VMEM((KC, 128), jnp.float32),
            pltpu.VMEM_SHARED((NPAD, 128), jnp.float32),
            pltpu.SemaphoreType.DMA,
        ],
    )
    return f(xlv, xrv, srcp, dstp, attr, z128)


def _sc_l1_messages(xlv, srcp, dstp, exh, denf, z128):
    nh = 4

    def body(xlv_r, srcp_r, dstp_r, exh_r, denf_r, z128_r, out_r,
             srcb, dstb, gsb, gdb, xlb, denb, exb, msgb, outs, sem):
        cid = lax.axis_index("c")
        sid = lax.axis_index("s")
        rsl = pl.ds(sid * RPT, RPT)
        pltpu.sync_copy(z128_r.at[rsl], outs.at[rsl])
        plsc.subcore_barrier()

        io = lax.iota(jnp.int32, 16)
        ebase0 = sid * EPT

        def chunk_body(k, carry):
            base = ebase0 + k * KC
            d1 = pltpu.async_copy(srcp_r.at[pl.ds(base, KC)], srcb, sem)
            d2 = pltpu.async_copy(dstp_r.at[pl.ds(base, KC)], dstb, sem)
            d3 = pltpu.async_copy(exh_r.at[cid, sid, k], exb, sem)
            d1.wait()
            d2.wait()
            for i in range(KC // 16):
                sl = pl.ds(16 * i, 16)
                gsb[sl] = srcb[sl] * 2 + cid
                gdb[sl] = dstb[sl] + cid * NPAD
            d1 = pltpu.async_copy(xlv_r.at[gsb], xlb, sem)
            d2 = pltpu.async_copy(denf_r.at[gdb], denb, sem)
            d1.wait()
            d2.wait()
            d3.wait()
            for g in range(KC // 16):
                exvs = [exb[pl.ds(h * KC + g * 16, 16)] for h in range(nh)]

                for j in range(16):
                    j2 = g * 16 + j
                    for h in range(nh):
                        asp = (_splat(exvs[h], j, io)
                               / (denb[j2, pl.ds(h * 32, 16)] + 1e-16))
                        for q in range(2):
                            sl = pl.ds((2 * h + q) * 16, 16)
                            msgb[j2, sl] = xlb[j2, sl] * asp
            pltpu.sync_copy(msgb, outs.at[dstb], add=True)
            return carry

        lax.fori_loop(0, NKC, chunk_body, 0)
        plsc.subcore_barrier()
        pltpu.sync_copy(outs.at[rsl], out_r.at[cid, rsl])

    f = pl.kernel(
        body,
        out_type=[jax.ShapeDtypeStruct((NC, NPAD, 128), jnp.float32)],
        mesh=_mesh,
        scratch_types=[
            pltpu.VMEM((KC,), jnp.int32),
            pltpu.VMEM((KC,), jnp.int32),
            pltpu.VMEM((KC,), jnp.int32),
            pltpu.VMEM((KC,), jnp.int32),
            pltpu.VMEM((KC, 128), jnp.float32),
            pltpu.VMEM((KC, 128), jnp.float32),
            pltpu.VMEM((4 * KC,), jnp.float32),
            pltpu.VMEM((KC, 128), jnp.float32),
            pltpu.VMEM_SHARED((NPAD, 128), jnp.float32),
            pltpu.SemaphoreType.DMA,
        ],
    )
    return f(xlv, srcp, dstp, exh, denf, z128)


def _sc_l2_plogits(xlv, xrv, srcp, dstp, attr):
    def body(xlv_r, xrv_r, srcp_r, dstp_r, attr_r, ph_r,
             srcb, dstb, gsb, gdb, xlb, xrb, attb, pb, sem):
        cid = lax.axis_index("c")
        sid = lax.axis_index("s")
        pltpu.sync_copy(attr_r.at[cid], attb)
        io = lax.iota(jnp.int32, 16)
        attv = [attb[pl.ds(16 * c, 16)] for c in range(8)]
        ebase0 = sid * EPT

        def chunk_body(k, carry):
            base = ebase0 + k * CHUNK
            d1 = pltpu.async_copy(srcp_r.at[pl.ds(base, CHUNK)], srcb, sem)
            d2 = pltpu.async_copy(dstp_r.at[pl.ds(base, CHUNK)], dstb, sem)
            d1.wait()
            d2.wait()
            for i in range(CHUNK // 16):
                sl = pl.ds(16 * i, 16)
                gsb[sl] = srcb[sl] * 2 + cid
                gdb[sl] = dstb[sl] * 2 + cid
            d1 = pltpu.async_copy(xlv_r.at[gsb], xlb, sem)
            d2 = pltpu.async_copy(xrv_r.at[gdb], xrb, sem)
            d1.wait()
            d2.wait()
            for g in range(CHUNK // 16):
                def logit_body(j, acc):
                    j2 = g * 16 + j
                    part = None
                    for ch in range(8):
                        sl = pl.ds(16 * ch, 16)
                        hv = xlb[j2, sl] + xrb[j2, sl]
                        ev = jnp.maximum(hv, SLOPE * hv)
                        t = ev * attv[ch]
                        part = t if part is None else part + t
                    tot = _lanesum(part, io)
                    return jnp.where(io == j, tot, acc)
                acc = lax.fori_loop(0, 16, logit_body, jnp.zeros((16,), jnp.float32))
                pb[pl.ds(g * 16, 16)] = acc
            pltpu.sync_copy(pb, ph_r.at[cid, pl.ds(base, CHUNK)])
            return carry

        lax.fori_loop(0, NCHUNKS, chunk_body, 0)

    f = pl.kernel(
        body,
        out_type=[jax.ShapeDtypeStruct((NC, EPAD), jnp.float32)],
        mesh=_mesh,
        scratch_types=[
            pltpu.VMEM((CHUNK,), jnp.int32),
            pltpu.VMEM((CHUNK,), jnp.int32),
            pltpu.VMEM((CHUNK,), jnp.int32),
            pltpu.VMEM((CHUNK,), jnp.int32),
            pltpu.VMEM((CHUNK, 128), jnp.float32),
            pltpu.VMEM((CHUNK, 128), jnp.float32),
            pltpu.VMEM((128,), jnp.float32),
            pltpu.VMEM((CHUNK,), jnp.float32),
            pltpu.SemaphoreType.DMA,
        ],
    )
    return f(xlv, xrv, srcp, dstp, attr)


def _sc_l2_combine(ph, dstp, z128):
    def body(ph_r, dstp_r, z128_r, ex2_r, den2_r,
             p0b, p1b, exb, dstb, exrow, dens, sem):
        cid = lax.axis_index("c")
        sid = lax.axis_index("s")
        rsl = pl.ds(sid * RPT, RPT)
        pltpu.sync_copy(z128_r.at[rsl], dens.at[rsl])
        plsc.subcore_barrier()
        ebase0 = sid * EPT

        def chunk_body(k, carry):
            base = ebase0 + k * CHUNK2
            d1 = pltpu.async_copy(ph_r.at[0, pl.ds(base, CHUNK2)], p0b, sem)
            d2 = pltpu.async_copy(ph_r.at[1, pl.ds(base, CHUNK2)], p1b, sem)
            d3 = pltpu.async_copy(dstp_r.at[pl.ds(base, CHUNK2)], dstb, sem)
            d1.wait()
            d2.wait()
            d3.wait()
            exvs = []
            for i in range(CHUNK2 // 16):
                sl = pl.ds(16 * i, 16)
                ev = jnp.exp(p0b[sl] + p1b[sl])
                exb[sl] = ev
                exvs.append(ev)
            io = lax.iota(jnp.int32, 16)
            for g in range(CHUNK2 // 16):
                exv = exvs[g]

                for j in range(16):
                    sp = _splat(exv, j, io)
                    exrow[g * 16 + j, pl.ds(0, 16)] = sp
            pltpu.sync_copy(exb, ex2_r.at[cid, pl.ds(base, CHUNK2)])
            pltpu.sync_copy(exrow, dens.at[dstb], add=True)
            return carry

        lax.fori_loop(0, NCHUNKS2, chunk_body, 0)
        plsc.subcore_barrier()
        pltpu.sync_copy(dens.at[rsl], den2_r.at[cid, rsl])

    f = pl.kernel(
        body,
        out_type=[jax.ShapeDtypeStruct((NC, EPAD), jnp.float32),
                  jax.ShapeDtypeStruct((NC, NPAD, 128), jnp.float32)],
        mesh=_mesh,
        scratch_types=[
            pltpu.VMEM((CHUNK2,), jnp.float32),
            pltpu.VMEM((CHUNK2,), jnp.float32),
            pltpu.VMEM((CHUNK2,), jnp.float32),
            pltpu.VMEM((CHUNK2,), jnp.int32),
            pltpu.VMEM((CHUNK2, 128), jnp.float32),
            pltpu.VMEM_SHARED((NPAD, 128), jnp.float32),
            pltpu.SemaphoreType.DMA,
        ],
    )
    return f(ph, dstp, z128)


def _sc_l2_messages(xlv, srcp, dstp, ex2, denf, z128):
    def body(xlv_r, srcp_r, dstp_r, ex2_r, denf_r, z128_r, out_r,
             srcb, dstb, gsb, gdb, xlb, denb, exb, msgb, outs, sem):
        cid = lax.axis_index("c")
        sid = lax.axis_index("s")
        rsl = pl.ds(sid * RPT, RPT)
        pltpu.sync_copy(z128_r.at[rsl], outs.at[rsl])
        plsc.subcore_barrier()
        io = lax.iota(jnp.int32, 16)
        ebase0 = sid * EPT

        def chunk_body(k, carry):
            base = ebase0 + k * KC
            d1 = pltpu.async_copy(srcp_r.at[pl.ds(base, KC)], srcb, sem)
            d2 = pltpu.async_copy(dstp_r.at[pl.ds(base, KC)], dstb, sem)
            d3 = pltpu.async_copy(ex2_r.at[cid, pl.ds(base, KC)], exb, sem)
            d1.wait()
            d2.wait()
            for i in range(KC // 16):
                sl = pl.ds(16 * i, 16)
                gsb[sl] = srcb[sl] * 2 + cid
                gdb[sl] = dstb[sl] + cid * NPAD
            d1 = pltpu.async_copy(xlv_r.at[gsb], xlb, sem)
            d2 = pltpu.async_copy(denf_r.at[gdb], denb, sem)
            d1.wait()
            d2.wait()
            d3.wait()
            for g in range(KC // 16):
                exv = exb[pl.ds(g * 16, 16)]

                for j in range(16):
                    j2 = g * 16 + j
                    asp = (_splat(exv, j, io)
                           / (denb[j2, pl.ds(0, 16)] + 1e-16))
                    for ch in range(8):
                        sl = pl.ds(16 * ch, 16)
                        msgb[j2, sl] = xlb[j2, sl] * asp
            pltpu.sync_copy(msgb, outs.at[dstb], add=True)
            return carry

        lax.fori_loop(0, NKC, chunk_body, 0)
        plsc.subcore_barrier()
        pltpu.sync_copy(outs.at[rsl], out_r.at[cid, rsl])

    f = pl.kernel(
        body,
        out_type=[jax.ShapeDtypeStruct((NC, NPAD, 128), jnp.float32)],
        mesh=_mesh,
        scratch_types=[
            pltpu.VMEM((KC,), jnp.int32),
            pltpu.VMEM((KC,), jnp.int32),
            pltpu.VMEM((KC,), jnp.int32),
            pltpu.VMEM((KC,), jnp.int32),
            pltpu.VMEM((KC, 128), jnp.float32),
            pltpu.VMEM((KC, 128), jnp.float32),
            pltpu.VMEM((KC,), jnp.float32),
            pltpu.VMEM((KC, 128), jnp.float32),
            pltpu.VMEM_SHARED((NPAD, 128), jnp.float32),
            pltpu.SemaphoreType.DMA,
        ],
    )
    return f(xlv, srcp, dstp, ex2, denf, z128)


_ROWS_BLK = 1000


def _tc_mm1(x, Wl, Wr):
    def body(x_ref, wl_ref, wr_ref, o1_ref, o2_ref):
        xv = x_ref[...]
        o1_ref[...] = jnp.dot(xv, wl_ref[...], preferred_element_type=jnp.float32)
        o2_ref[...] = jnp.dot(xv, wr_ref[...], preferred_element_type=jnp.float32)

    nb = N // _ROWS_BLK
    return pl.pallas_call(
        body,
        grid=(nb,),
        in_specs=[
            pl.BlockSpec((_ROWS_BLK, D), lambda i: (i, 0)),
            pl.BlockSpec((D, HID), lambda i: (0, 0)),
            pl.BlockSpec((D, HID), lambda i: (0, 0)),
        ],
        out_specs=[
            pl.BlockSpec((_ROWS_BLK, HID), lambda i: (i, 0)),
            pl.BlockSpec((_ROWS_BLK, HID), lambda i: (i, 0)),
        ],
        out_shape=[jax.ShapeDtypeStruct((N, HID), jnp.float32),
                   jax.ShapeDtypeStruct((N, HID), jnp.float32)],
    )(x, Wl, Wr)


def _tc_mm2(acc1, sg, off, Wl, Wr):
    def body(a_ref, sg_ref, off_ref, wl_ref, wr_ref, o1_ref, o2_ref):
        h = jnp.maximum(a_ref[...] * sg_ref[...] + off_ref[...], 0.0)
        o1_ref[...] = jnp.dot(h, wl_ref[...], preferred_element_type=jnp.float32)
        o2_ref[...] = jnp.dot(h, wr_ref[...], preferred_element_type=jnp.float32)

    nb = N // _ROWS_BLK
    return pl.pallas_call(
        body,
        grid=(nb,),
        in_specs=[
            pl.BlockSpec((_ROWS_BLK, HID), lambda i: (i, 0)),
            pl.BlockSpec((1, HID), lambda i: (0, 0)),
            pl.BlockSpec((1, HID), lambda i: (0, 0)),
            pl.BlockSpec((HID, HID), lambda i: (0, 0)),
            pl.BlockSpec((HID, HID), lambda i: (0, 0)),
        ],
        out_specs=[
            pl.BlockSpec((_ROWS_BLK, HID), lambda i: (i, 0)),
            pl.BlockSpec((_ROWS_BLK, HID), lambda i: (i, 0)),
        ],
        out_shape=[jax.ShapeDtypeStruct((N, HID), jnp.float32),
                   jax.ShapeDtypeStruct((N, HID), jnp.float32)],
    )(acc1, sg, off, Wl, Wr)


def _tc_final(acc2, sg, off, gW1, gb1, gW2t, batch2, fW1, fb1, fW2t, fb2):
    def body(a_ref, sg_ref, off_ref, gw1_ref, gb1_ref, gw2_ref, b_ref,
             fw1_ref, fb1_ref, fw2_ref, fb2_ref, o_ref):
        h2 = jnp.maximum(a_ref[...] * sg_ref[...] + off_ref[...], 0.0)
        t = jnp.tanh(jnp.dot(h2, gw1_ref[...],
                             preferred_element_type=jnp.float32) + gb1_ref[...])
        gate = lax.dot_general(gw2_ref[...], t, (((1,), (1,)), ((), ())),
                               preferred_element_type=jnp.float32)
        gmax = jnp.max(gate)
        gate2 = jnp.exp(gate - gmax)
        mask = lax.broadcasted_iota(jnp.int32, (G, N), 0) == b_ref[...]
        ge = jnp.where(mask, gate2, 0.0)
        gdt = lax.dot_general(jnp.ones((1, N), jnp.float32), ge,
                              (((1,), (1,)), ((), ())),
                              preferred_element_type=jnp.float32)
        poolt = lax.dot_general(h2, ge, (((0,), (1,)), ((), ())),
                                preferred_element_type=jnp.float32)
        poolt = poolt * (1.0 / (gdt + 1e-16))
        r = jnp.maximum(
            lax.dot_general(poolt, fw1_ref[...], (((0,), (0,)), ((), ())),
                            preferred_element_type=jnp.float32)
            + fb1_ref[...], 0.0)
        o_ref[...] = lax.dot_general(fw2_ref[...], r, (((1,), (1,)), ((), ())),
                                     preferred_element_type=jnp.float32) + fb2_ref[0, 0]

    out_t = pl.pallas_call(
        body,
        out_shape=jax.ShapeDtypeStruct((1, G), jnp.float32),
    )(acc2, sg, off, gW1, gb1, gW2t, batch2, fW1, fb1, fW2t, fb2)
    return out_t.reshape(G, 1)


def kernel(x, edge_index, batch, Wl1, Wr1, att1, b1, Wl2, Wr2, att2, b2,
           bn_g, bn_b, gW1, gb1, gW2, fW1, fb1, fW2, fb2):
    src = edge_index[0]
    dst = edge_index[1]
    npad_e = EPAD - E
    srcp = jnp.concatenate([src, jnp.zeros((npad_e,), jnp.int32)])
    dstp = jnp.concatenate([dst, jnp.full((npad_e,), N, jnp.int32)])
    z128 = jnp.zeros((NPAD, 128), jnp.float32)

    inv = 1.0 / jnp.sqrt(1.0 + BN_EPS)
    sg = (bn_g * inv).reshape(1, HID)
    off1 = (b1 * sg[0] + bn_b).reshape(1, HID)
    off2 = (b2 * sg[0] + bn_b).reshape(1, HID)

    xl1, xr1 = _tc_mm1(x, Wl1, Wr1)
    xlv = xl1.reshape(2 * N, 128)
    xrv = xr1.reshape(2 * N, 128)
    exh, denh = _sc_l1_logits(xlv, xrv, srcp, dstp,
                              att1.reshape(NC, 128), z128)
    denf = denh.reshape(NC * NPAD, 128)
    (o1h,) = _sc_l1_messages(xlv, srcp, dstp, exh, denf, z128)
    acc1 = jnp.concatenate([o1h[0, :N], o1h[1, :N]], axis=1)

    xl2, xr2 = _tc_mm2(acc1, sg, off1, Wl2, Wr2)
    xl2v = xl2.reshape(2 * N, 128)
    xr2v = xr2.reshape(2 * N, 128)
    (ph,) = _sc_l2_plogits(xl2v, xr2v, srcp, dstp, att2.reshape(NC, 128))
    ex2, den2 = _sc_l2_combine(ph, dstp, z128)
    den2f = den2.reshape(NC * NPAD, 128)
    (o2h,) = _sc_l2_messages(xl2v, srcp, dstp, ex2, den2f, z128)
    acc2 = jnp.concatenate([o2h[0, :N], o2h[1, :N]], axis=1)

    out = _tc_final(acc2, sg, off2, gW1, gb1.reshape(1, HID),
                    gW2.reshape(1, HID), batch.reshape(1, N),
                    fW1, fb1.reshape(1, 100), fW2.reshape(1, 100),
                    fb2.reshape(1, 1))
    return out
